# Initial kernel scaffold; baseline (speedup 1.0000x reference)
import jax
import jax.numpy as jnp
from jax.experimental import pallas as pl


def kernel(x, edge_index, in_w, in_b, edge_w1, edge_b1, edge_w2, edge_b2, conv_w1, conv_b1, conv_w2, conv_b2, norm_g, norm_b, final_w, final_b, out_g, out_b):
    raise NotImplementedError("write your pallas kernel here")


# speedup vs baseline: 1.9730x; 1.9730x over previous
import math

import jax
import jax.numpy as jnp
from jax import lax
from jax.experimental import pallas as pl
from jax.experimental.pallas import tpu as pltpu
from jax.experimental.pallas import tpu_sc as plsc

_N = 10000
_E = 320000
_H = 128
_NC = 2
_NS = 16
_NPC = _N // _NS
_E2 = _E // _NC
_OCH = 2000
_NOCH = _E2 // _OCH
_FL = 2048
_BUF = 4096
_CAP = 161920
_B = 64
_NACC = 632
_BREL = 400
_EPW = _E // (_NC * _NS)
_NCHREL = _EPW // _BREL

_mesh = plsc.VectorSubcoreMesh(core_axis_name="c", subcore_axis_name="s")

_SQRT2 = math.sqrt(2.0)


def _rel_body(posx_h, posy_h, row_h, col_h, zer8_h, rel8_h,
              posx_v, posy_v, rowb, colb, rel8b, sem):
    c = lax.axis_index("c")
    s = lax.axis_index("s")
    wid = c * _NS + s
    pltpu.sync_copy(posx_h, posx_v)
    pltpu.sync_copy(posy_h, posy_v)
    pltpu.sync_copy(zer8_h, rel8b)
    base0 = wid * _EPW

    @pl.loop(0, _NCHREL)
    def _chunk(ci):
        base = pl.multiple_of(base0 + ci * _BREL, 8)
        pltpu.sync_copy(row_h.at[pl.ds(base, _BREL)], rowb)
        pltpu.sync_copy(col_h.at[pl.ds(base, _BREL)], colb)
        lane = lax.iota(jnp.int32, 16)
        zero16 = jnp.zeros((16,), jnp.int32)
        one16 = jnp.ones((16,), jnp.int32)
        for g in range(_BREL // 16):
            sl = pl.ds(g * 16, 16)
            rv = rowb[sl]
            cv = colb[sl]
            rx = plsc.load_gather(posx_v, [cv]) - plsc.load_gather(posx_v, [rv])
            ry = plsc.load_gather(posy_v, [cv]) - plsc.load_gather(posy_v, [rv])
            ridx = lane + (g * 16)
            plsc.store_scatter(rel8b, [ridx, zero16], rx)
            plsc.store_scatter(rel8b, [ridx, one16], ry)
        pltpu.sync_copy(rel8b, rel8_h.at[pl.ds(base, _BREL)])


def _rel_pos_sc(posx, posy, row, col, zer8):
    f = pl.kernel(
        _rel_body,
        out_type=jax.ShapeDtypeStruct((_E, 8), jnp.float32),
        mesh=_mesh,
        compiler_params=pltpu.CompilerParams(needs_layout_passes=False),
        scratch_types=[
            pltpu.VMEM((_N,), jnp.float32),
            pltpu.VMEM((_N,), jnp.float32),
            pltpu.VMEM((_BREL,), jnp.int32),
            pltpu.VMEM((_BREL,), jnp.int32),
            pltpu.VMEM((_BREL, 8), jnp.float32),
            pltpu.SemaphoreType.DMA,
        ],
    )
    return f(posx, posy, row, col, zer8)


def _bucket_body(row_h, col_h, eidL, rowL, colL, cnts_h,
                 crow, ccol, beid, brow, bcol, cntv, sem):
    c = lax.axis_index("c")
    s = lax.axis_index("s")
    w = c * _NS + s
    lo = s * _NPC
    base0 = c * _E2
    lane = lax.iota(jnp.int32, 16)

    def outer(i, carry):
        off, cnt = carry
        ebase = pl.multiple_of(base0 + i * _OCH, 8)
        pltpu.sync_copy(col_h.at[pl.ds(ebase, _OCH)], ccol)
        pltpu.sync_copy(row_h.at[pl.ds(ebase, _OCH)], crow)

        def inner(j, cnt):
            sl = pl.ds(j * 16, 16)
            col16 = ccol[sl]
            row16 = crow[sl]
            msk = (col16 >= lo) & (col16 < lo + _NPC)
            eid16 = jnp.full((16,), ebase, jnp.int32) + j * 16 + lane
            pos = cnt + plsc.cumsum(msk.astype(jnp.int32)) - 1
            plsc.store_scatter(beid, [pos], eid16, mask=msk)
            plsc.store_scatter(brow, [pos], row16, mask=msk)
            plsc.store_scatter(bcol, [pos], col16 - lo, mask=msk)
            return cnt + jnp.sum(msk.astype(jnp.int32))

        cnt = lax.fori_loop(0, _OCH // 16, inner, cnt)
        wb = pl.multiple_of(w * _CAP + off, 128)
        pltpu.sync_copy(beid.at[pl.ds(0, _FL)], eidL.at[pl.ds(wb, _FL)])
        pltpu.sync_copy(brow.at[pl.ds(0, _FL)], rowL.at[pl.ds(wb, _FL)])
        pltpu.sync_copy(bcol.at[pl.ds(0, _FL)], colL.at[pl.ds(wb, _FL)])
        flushed = (cnt >= _FL).astype(jnp.int32)
        fmask = jnp.full((16,), flushed, jnp.int32) > 0

        @pl.loop(0, _FL // 16)
        def _shift(j):
            d = pl.ds(j * 16, 16)
            u = pl.ds(_FL + j * 16, 16)
            beid[d] = jnp.where(fmask, beid[u], beid[d])
            brow[d] = jnp.where(fmask, brow[u], brow[d])
            bcol[d] = jnp.where(fmask, bcol[u], bcol[d])

        return (off + _FL * flushed, cnt - _FL * flushed)

    off, cnt = lax.fori_loop(0, _NOCH, outer, (0, 0))

    zero16 = jnp.zeros((16,), jnp.int32)
    spill16 = jnp.full((16,), _NPC, jnp.int32)
    for k in range(9):
        beid[pl.ds(cnt + k * 16, 16)] = zero16
        brow[pl.ds(cnt + k * 16, 16)] = zero16
        bcol[pl.ds(cnt + k * 16, 16)] = spill16
    wb = pl.multiple_of(w * _CAP + off, 128)
    pltpu.sync_copy(beid.at[pl.ds(0, _FL + 128)],
                    eidL.at[pl.ds(wb, _FL + 128)])
    pltpu.sync_copy(brow.at[pl.ds(0, _FL + 128)],
                    rowL.at[pl.ds(wb, _FL + 128)])
    pltpu.sync_copy(bcol.at[pl.ds(0, _FL + 128)],
                    colL.at[pl.ds(wb, _FL + 128)])
    nch = (off + cnt + _B - 1) // _B
    cntv[...] = jnp.full((16,), nch, jnp.int32)
    pltpu.sync_copy(cntv, cnts_h.at[pl.ds(pl.multiple_of(w * 16, 16), 16)])


def _bucket_sc(row, col):
    f = pl.kernel(
        _bucket_body,
        out_type=[
            jax.ShapeDtypeStruct((_NC * _NS * _CAP,), jnp.int32),
            jax.ShapeDtypeStruct((_NC * _NS * _CAP,), jnp.int32),
            jax.ShapeDtypeStruct((_NC * _NS * _CAP,), jnp.int32),
            jax.ShapeDtypeStruct((_NC * _NS * 16,), jnp.int32),
        ],
        mesh=_mesh,
        compiler_params=pltpu.CompilerParams(needs_layout_passes=False),
        scratch_types=[
            pltpu.VMEM((_OCH,), jnp.int32),
            pltpu.VMEM((_OCH,), jnp.int32),
            pltpu.VMEM((_BUF,), jnp.int32),
            pltpu.VMEM((_BUF,), jnp.int32),
            pltpu.VMEM((_BUF,), jnp.int32),
            pltpu.VMEM((16,), jnp.int32),
            pltpu.SemaphoreType.DMA,
        ],
    )
    return f(row, col)


def _msg_body(h_h, ea_h, eidL, rowL, colL, cnts_h, out_h,
              eidb, rowb, colb, hab, eab, cntv, acc, sem1, sem2):
    c = lax.axis_index("c")
    s = lax.axis_index("s")
    w = c * _NS + s
    zero16 = jnp.zeros((16,), jnp.float32)

    @pl.loop(0, _NACC)
    def _z(r):
        for g in range(_H // 16):
            acc[r, pl.ds(g * 16, 16)] = zero16

    pltpu.sync_copy(cnts_h.at[pl.ds(pl.multiple_of(w * 16, 16), 16)], cntv)
    nch = cntv[...][0]

    def chunk(ci, _):
        base = pl.multiple_of(w * _CAP + ci * _B, 8)
        pltpu.sync_copy(eidL.at[pl.ds(base, _B)], eidb)
        pltpu.sync_copy(rowL.at[pl.ds(base, _B)], rowb)
        pltpu.sync_copy(colL.at[pl.ds(base, _B)], colb)
        cp1 = pltpu.async_copy(h_h.at[rowb], hab, sem1)
        cp2 = pltpu.async_copy(ea_h.at[eidb], eab, sem2)
        cp1.wait()
        cp2.wait()

        @pl.loop(0, _B // 16)
        def _eg(j):
            col16 = colb[pl.ds(j * 16, 16)]
            for t in range(16):
                k = j * 16 + t
                cl = col16[t]
                for g in range(_H // 16):
                    sl = pl.ds(g * 16, 16)
                    m = jnp.maximum(hab[k, sl] + eab[k, sl], 0.0)
                    plsc.addupdate(acc.at[cl, sl], m)

        return 0

    lax.fori_loop(0, nch, chunk, 0)
    pltpu.sync_copy(acc, out_h.at[c, s])


def _msg_sc(h, ea, eidL, rowL, colL, cnts):
    f = pl.kernel(
        _msg_body,
        out_type=jax.ShapeDtypeStruct((_NC, _NS, _NACC, _H), jnp.float32),
        mesh=_mesh,
        compiler_params=pltpu.CompilerParams(needs_layout_passes=False),
        scratch_types=[
            pltpu.VMEM((_B,), jnp.int32),
            pltpu.VMEM((_B,), jnp.int32),
            pltpu.VMEM((_B,), jnp.int32),
            pltpu.VMEM((_B, _H), jnp.float32),
            pltpu.VMEM((_B, _H), jnp.float32),
            pltpu.VMEM((16,), jnp.int32),
            pltpu.VMEM((_NACC, _H), jnp.float32),
            pltpu.SemaphoreType.DMA,
            pltpu.SemaphoreType.DMA,
        ],
    )
    return f(h, ea, eidL, rowL, colL, cnts)


_NB = 400
_NGRID = _N // _NB
_EB = 640
_EGRID = _E // _EB


def _h0_body(x_ref, w_ref, b_ref, o_ref):
    o_ref[...] = (jnp.dot(x_ref[...], w_ref[...],
                          preferred_element_type=jnp.float32) + b_ref[...])


def _h0_tc(x, in_w, in_b):
    return pl.pallas_call(
        _h0_body,
        grid=(_NGRID,),
        in_specs=[
            pl.BlockSpec((_NB, _H), lambda i: (i, 0)),
            pl.BlockSpec((_H, _H), lambda i: (0, 0)),
            pl.BlockSpec((1, _H), lambda i: (0, 0)),
        ],
        out_specs=pl.BlockSpec((_NB, _H), lambda i: (i, 0)),
        out_shape=jax.ShapeDtypeStruct((_N, _H), jnp.float32),
    )(x, in_w, in_b.reshape(1, _H))


def _edge_body(r8_ref, w1_ref, b1_ref, w2_ref, b2_ref, o_ref):
    a1 = jnp.maximum(
        jnp.dot(r8_ref[...], w1_ref[...],
                preferred_element_type=jnp.float32) + b1_ref[...], 0.0)
    o_ref[...] = (jnp.dot(a1, w2_ref[...],
                          preferred_element_type=jnp.float32) + b2_ref[...])


def _edge_tc(rel8, w1pad, b1, w2, b2):
    return pl.pallas_call(
        _edge_body,
        grid=(_EGRID,),
        in_specs=[
            pl.BlockSpec((_EB, 8), lambda i: (i, 0)),
            pl.BlockSpec((8, _H), lambda i: (0, 0)),
            pl.BlockSpec((1, _H), lambda i: (0, 0)),
            pl.BlockSpec((_H, _H), lambda i: (0, 0)),
            pl.BlockSpec((1, _H), lambda i: (0, 0)),
        ],
        out_specs=pl.BlockSpec((_EB, _H), lambda i: (i, 0)),
        out_shape=jax.ShapeDtypeStruct((_E, _H), jnp.float32),
    )(rel8, w1pad, b1.reshape(1, _H), w2, b2.reshape(1, _H))


def _ln(t, g, b):
    mu = jnp.mean(t, axis=-1, keepdims=True)
    var = jnp.mean((t - mu) * (t - mu), axis=-1, keepdims=True)
    return (t - mu) * lax.rsqrt(var + 1e-5) * g + b


def _node_body(h_ref, a0_ref, a1_ref, w1_ref, b1_ref, w2_ref, b2_ref,
               g_ref, bb_ref, o_ref):
    z0 = a0_ref[...] + a1_ref[...] + h_ref[...]
    t = jnp.maximum(
        jnp.dot(z0, w1_ref[...], preferred_element_type=jnp.float32)
        + b1_ref[...], 0.0)
    t = (jnp.dot(t, w2_ref[...], preferred_element_type=jnp.float32)
         + b2_ref[...])
    t = _ln(t, g_ref[...], bb_ref[...])
    t = t * 0.5 * (1.0 + lax.erf(t / _SQRT2))
    o_ref[...] = t + h_ref[...]


def _node_tc(h, agg2, w1, b1, w2, b2, g, b):
    return pl.pallas_call(
        _node_body,
        grid=(_NGRID,),
        in_specs=[
            pl.BlockSpec((_NB, _H), lambda i: (i, 0)),
            pl.BlockSpec((_NB, _H), lambda i: (i, 0)),
            pl.BlockSpec((_NB, _H), lambda i: (i + _NGRID, 0)),
            pl.BlockSpec((_H, _H), lambda i: (0, 0)),
            pl.BlockSpec((1, _H), lambda i: (0, 0)),
            pl.BlockSpec((_H, _H), lambda i: (0, 0)),
            pl.BlockSpec((1, _H), lambda i: (0, 0)),
            pl.BlockSpec((1, _H), lambda i: (0, 0)),
            pl.BlockSpec((1, _H), lambda i: (0, 0)),
        ],
        out_specs=pl.BlockSpec((_NB, _H), lambda i: (i, 0)),
        out_shape=jax.ShapeDtypeStruct((_N, _H), jnp.float32),
    )(h, agg2, agg2, w1, b1.reshape(1, _H), w2, b2.reshape(1, _H),
      g.reshape(1, _H), b.reshape(1, _H))


def _final_body(h_ref, w_ref, b_ref, g_ref, bb_ref, o_ref):
    t = (jnp.dot(h_ref[...], w_ref[...], preferred_element_type=jnp.float32)
         + b_ref[...] + h_ref[...])
    o_ref[...] = _ln(t, g_ref[...], bb_ref[...])


def _final_tc(h, w, b, g, bb):
    return pl.pallas_call(
        _final_body,
        grid=(_NGRID,),
        in_specs=[
            pl.BlockSpec((_NB, _H), lambda i: (i, 0)),
            pl.BlockSpec((_H, _H), lambda i: (0, 0)),
            pl.BlockSpec((1, _H), lambda i: (0, 0)),
            pl.BlockSpec((1, _H), lambda i: (0, 0)),
            pl.BlockSpec((1, _H), lambda i: (0, 0)),
        ],
        out_specs=pl.BlockSpec((_NB, _H), lambda i: (i, 0)),
        out_shape=jax.ShapeDtypeStruct((_N, _H), jnp.float32),
    )(h, w, b.reshape(1, _H), g.reshape(1, _H), bb.reshape(1, _H))


def kernel(x, edge_index, in_w, in_b, edge_w1, edge_b1, edge_w2, edge_b2,
           conv_w1, conv_b1, conv_w2, conv_b2, norm_g, norm_b,
           final_w, final_b, out_g, out_b):
    row = edge_index[0]
    col = edge_index[1]
    posx = x[:, 0]
    posy = x[:, 1]
    zer8 = jnp.zeros((_BREL, 8), jnp.float32)
    w1pad = jnp.concatenate(
        [edge_w1, jnp.zeros((6, _H), jnp.float32)], axis=0)

    h = _h0_tc(x, in_w, in_b)
    rel8 = _rel_pos_sc(posx, posy, row, col, zer8)
    ea = _edge_tc(rel8, w1pad, edge_b1, edge_w2, edge_b2)
    eidL, rowL, colL, cnts = _bucket_sc(row, col)
    for i in range(4):
        agg4 = _msg_sc(h, ea, eidL, rowL, colL, cnts)
        agg2 = agg4[:, :, :_NPC, :].reshape(_NC * _N, _H)
        h = _node_tc(h, agg2, conv_w1[i], conv_b1[i], conv_w2[i], conv_b2[i],
                     norm_g[i], norm_b[i])
    return _final_tc(h, final_w, final_b, out_g, out_b)


# speedup vs baseline: 2.3703x; 1.2014x over previous
import math

import jax
import jax.numpy as jnp
from jax import lax
from jax.experimental import pallas as pl
from jax.experimental.pallas import tpu as pltpu
from jax.experimental.pallas import tpu_sc as plsc

_N = 10000
_E = 320000
_H = 128
_NC = 2
_NS = 16
_NPC = _N // _NS
_E2 = _E // _NC
_OCH = 2000
_NOCH = _E2 // _OCH
_FL = 2048
_BUF = 4096
_CAP = 161920
_B = 64
_NACC = 632
_BREL = 400
_EPW = _E // (_NC * _NS)
_NCHREL = _EPW // _BREL

_mesh = plsc.VectorSubcoreMesh(core_axis_name="c", subcore_axis_name="s")

_SQRT2 = math.sqrt(2.0)


def _rel_body(posx_h, posy_h, row_h, col_h, zer8_h, rel8_h,
              posx_v, posy_v, rowb, colb, rel8b, sem):
    c = lax.axis_index("c")
    s = lax.axis_index("s")
    wid = c * _NS + s
    pltpu.sync_copy(posx_h, posx_v)
    pltpu.sync_copy(posy_h, posy_v)
    pltpu.sync_copy(zer8_h, rel8b)
    base0 = wid * _EPW

    @pl.loop(0, _NCHREL)
    def _chunk(ci):
        base = pl.multiple_of(base0 + ci * _BREL, 8)
        pltpu.sync_copy(row_h.at[pl.ds(base, _BREL)], rowb)
        pltpu.sync_copy(col_h.at[pl.ds(base, _BREL)], colb)
        lane = lax.iota(jnp.int32, 16)
        zero16 = jnp.zeros((16,), jnp.int32)
        one16 = jnp.ones((16,), jnp.int32)
        for g in range(_BREL // 16):
            sl = pl.ds(g * 16, 16)
            rv = rowb[sl]
            cv = colb[sl]
            rx = plsc.load_gather(posx_v, [cv]) - plsc.load_gather(posx_v, [rv])
            ry = plsc.load_gather(posy_v, [cv]) - plsc.load_gather(posy_v, [rv])
            ridx = lane + (g * 16)
            plsc.store_scatter(rel8b, [ridx, zero16], rx)
            plsc.store_scatter(rel8b, [ridx, one16], ry)
        pltpu.sync_copy(rel8b, rel8_h.at[pl.ds(base, _BREL)])


def _rel_pos_sc(posx, posy, row, col, zer8):
    f = pl.kernel(
        _rel_body,
        out_type=jax.ShapeDtypeStruct((_E, 8), jnp.float32),
        mesh=_mesh,
        compiler_params=pltpu.CompilerParams(needs_layout_passes=False),
        scratch_types=[
            pltpu.VMEM((_N,), jnp.float32),
            pltpu.VMEM((_N,), jnp.float32),
            pltpu.VMEM((_BREL,), jnp.int32),
            pltpu.VMEM((_BREL,), jnp.int32),
            pltpu.VMEM((_BREL, 8), jnp.float32),
            pltpu.SemaphoreType.DMA,
        ],
    )
    return f(posx, posy, row, col, zer8)


def _bucket_body(row_h, col_h, eidL, rowL, colL, cnts_h,
                 crow, ccol, beid, brow, bcol, cntv, sem):
    c = lax.axis_index("c")
    s = lax.axis_index("s")
    w = c * _NS + s
    lo = s * _NPC
    base0 = c * _E2
    lane = lax.iota(jnp.int32, 16)

    def outer(i, carry):
        off, cnt = carry
        ebase = pl.multiple_of(base0 + i * _OCH, 8)
        pltpu.sync_copy(col_h.at[pl.ds(ebase, _OCH)], ccol)
        pltpu.sync_copy(row_h.at[pl.ds(ebase, _OCH)], crow)

        def inner(j, cnt):
            sl = pl.ds(j * 16, 16)
            col16 = ccol[sl]
            row16 = crow[sl]
            msk = (col16 >= lo) & (col16 < lo + _NPC)
            eid16 = jnp.full((16,), ebase, jnp.int32) + j * 16 + lane
            pos = cnt + plsc.cumsum(msk.astype(jnp.int32)) - 1
            plsc.store_scatter(beid, [pos], eid16, mask=msk)
            plsc.store_scatter(brow, [pos], row16, mask=msk)
            plsc.store_scatter(bcol, [pos], col16 - lo, mask=msk)
            return cnt + jnp.sum(msk.astype(jnp.int32))

        cnt = lax.fori_loop(0, _OCH // 16, inner, cnt)
        wb = pl.multiple_of(w * _CAP + off, 128)
        pltpu.sync_copy(beid.at[pl.ds(0, _FL)], eidL.at[pl.ds(wb, _FL)])
        pltpu.sync_copy(brow.at[pl.ds(0, _FL)], rowL.at[pl.ds(wb, _FL)])
        pltpu.sync_copy(bcol.at[pl.ds(0, _FL)], colL.at[pl.ds(wb, _FL)])
        flushed = (cnt >= _FL).astype(jnp.int32)
        fmask = jnp.full((16,), flushed, jnp.int32) > 0

        @pl.loop(0, _FL // 16)
        def _shift(j):
            d = pl.ds(j * 16, 16)
            u = pl.ds(_FL + j * 16, 16)
            beid[d] = jnp.where(fmask, beid[u], beid[d])
            brow[d] = jnp.where(fmask, brow[u], brow[d])
            bcol[d] = jnp.where(fmask, bcol[u], bcol[d])

        return (off + _FL * flushed, cnt - _FL * flushed)

    off, cnt = lax.fori_loop(0, _NOCH, outer, (0, 0))

    zero16 = jnp.zeros((16,), jnp.int32)
    spill16 = jnp.full((16,), _NPC, jnp.int32)
    for k in range(9):
        beid[pl.ds(cnt + k * 16, 16)] = zero16
        brow[pl.ds(cnt + k * 16, 16)] = zero16
        bcol[pl.ds(cnt + k * 16, 16)] = spill16
    wb = pl.multiple_of(w * _CAP + off, 128)
    pltpu.sync_copy(beid.at[pl.ds(0, _FL + 128)],
                    eidL.at[pl.ds(wb, _FL + 128)])
    pltpu.sync_copy(brow.at[pl.ds(0, _FL + 128)],
                    rowL.at[pl.ds(wb, _FL + 128)])
    pltpu.sync_copy(bcol.at[pl.ds(0, _FL + 128)],
                    colL.at[pl.ds(wb, _FL + 128)])
    nch = (off + cnt + _B - 1) // _B
    cntv[...] = jnp.full((16,), nch, jnp.int32)
    pltpu.sync_copy(cntv, cnts_h.at[pl.ds(pl.multiple_of(w * 16, 16), 16)])


def _bucket_sc(row, col):
    f = pl.kernel(
        _bucket_body,
        out_type=[
            jax.ShapeDtypeStruct((_NC * _NS * _CAP,), jnp.int32),
            jax.ShapeDtypeStruct((_NC * _NS * _CAP,), jnp.int32),
            jax.ShapeDtypeStruct((_NC * _NS * _CAP,), jnp.int32),
            jax.ShapeDtypeStruct((_NC * _NS * 16,), jnp.int32),
        ],
        mesh=_mesh,
        compiler_params=pltpu.CompilerParams(needs_layout_passes=False),
        scratch_types=[
            pltpu.VMEM((_OCH,), jnp.int32),
            pltpu.VMEM((_OCH,), jnp.int32),
            pltpu.VMEM((_BUF,), jnp.int32),
            pltpu.VMEM((_BUF,), jnp.int32),
            pltpu.VMEM((_BUF,), jnp.int32),
            pltpu.VMEM((16,), jnp.int32),
            pltpu.SemaphoreType.DMA,
        ],
    )
    return f(row, col)


def _msg_body(h_h, ea_h, eidL, rowL, colL, cnts_h, out_h,
              eidb0, rowb0, colb0, hab0, eab0,
              eidb1, rowb1, colb1, hab1, eab1,
              cntv, acc, sem10, sem20, sem11, sem21):
    c = lax.axis_index("c")
    s = lax.axis_index("s")
    w = c * _NS + s
    zero16 = jnp.zeros((16,), jnp.float32)
    slots = ((eidb0, rowb0, colb0, hab0, eab0, sem10, sem20),
             (eidb1, rowb1, colb1, hab1, eab1, sem11, sem21))

    @pl.loop(0, _NACC)
    def _z(r):
        for g in range(_H // 16):
            acc[r, pl.ds(g * 16, 16)] = zero16

    pltpu.sync_copy(cnts_h.at[pl.ds(pl.multiple_of(w * 16, 16), 16)], cntv)
    nch = cntv[...][0]

    def _issue(sl, ci):
        eidb, rowb, colb, hab, eab, s1, s2 = slots[sl]
        base = pl.multiple_of(w * _CAP, 8) + ci * _B
        pltpu.sync_copy(eidL.at[pl.ds(base, _B)], eidb)
        pltpu.sync_copy(rowL.at[pl.ds(base, _B)], rowb)
        pltpu.sync_copy(colL.at[pl.ds(base, _B)], colb)
        pltpu.async_copy(h_h.at[rowb], hab, s1)
        pltpu.async_copy(ea_h.at[eidb], eab, s2)

    def _drain_compute(sl):
        eidb, rowb, colb, hab, eab, s1, s2 = slots[sl]
        pltpu.make_async_copy(h_h.at[rowb], hab, s1).wait()
        pltpu.make_async_copy(ea_h.at[eidb], eab, s2).wait()

        @pl.loop(0, _B // 16)
        def _eg(j):
            col16 = colb[pl.ds(j * 16, 16)]
            for t in range(16):
                k = j * 16 + t
                cl = col16[t]
                for g in range(_H // 16):
                    sl2 = pl.ds(g * 16, 16)
                    m = jnp.maximum(hab[k, sl2] + eab[k, sl2], 0.0)
                    plsc.addupdate(acc.at[cl, sl2], m)

    @pl.when(nch > 0)
    def _prol():
        _issue(0, 0)

    def pair(g, _):
        for b in range(2):
            ci = 2 * g + b

            @pl.when(ci + 1 < nch)
            def _i():
                _issue(1 - b, ci + 1)

            @pl.when(ci < nch)
            def _c():
                _drain_compute(b)

        return 0

    lax.fori_loop(0, (nch + 1) // 2, pair, 0)
    pltpu.sync_copy(acc, out_h.at[c, s])


def _msg_sc(h, ea, eidL, rowL, colL, cnts):
    f = pl.kernel(
        _msg_body,
        out_type=jax.ShapeDtypeStruct((_NC, _NS, _NACC, _H), jnp.float32),
        mesh=_mesh,
        compiler_params=pltpu.CompilerParams(needs_layout_passes=False),
        scratch_types=[
            pltpu.VMEM((_B,), jnp.int32),
            pltpu.VMEM((_B,), jnp.int32),
            pltpu.VMEM((_B,), jnp.int32),
            pltpu.VMEM((_B, _H), jnp.float32),
            pltpu.VMEM((_B, _H), jnp.float32),
            pltpu.VMEM((_B,), jnp.int32),
            pltpu.VMEM((_B,), jnp.int32),
            pltpu.VMEM((_B,), jnp.int32),
            pltpu.VMEM((_B, _H), jnp.float32),
            pltpu.VMEM((_B, _H), jnp.float32),
            pltpu.VMEM((16,), jnp.int32),
            pltpu.VMEM((_NACC, _H), jnp.float32),
            pltpu.SemaphoreType.DMA,
            pltpu.SemaphoreType.DMA,
            pltpu.SemaphoreType.DMA,
            pltpu.SemaphoreType.DMA,
        ],
    )
    return f(h, ea, eidL, rowL, colL, cnts)


_NB = 400
_NGRID = _N // _NB
_EB = 640
_EGRID = _E // _EB


def _h0_body(x_ref, w_ref, b_ref, o_ref):
    o_ref[...] = (jnp.dot(x_ref[...], w_ref[...],
                          preferred_element_type=jnp.float32) + b_ref[...])


def _h0_tc(x, in_w, in_b):
    return pl.pallas_call(
        _h0_body,
        grid=(_NGRID,),
        in_specs=[
            pl.BlockSpec((_NB, _H), lambda i: (i, 0)),
            pl.BlockSpec((_H, _H), lambda i: (0, 0)),
            pl.BlockSpec((1, _H), lambda i: (0, 0)),
        ],
        out_specs=pl.BlockSpec((_NB, _H), lambda i: (i, 0)),
        out_shape=jax.ShapeDtypeStruct((_N, _H), jnp.float32),
    )(x, in_w, in_b.reshape(1, _H))


def _edge_body(r8_ref, w1_ref, b1_ref, w2_ref, b2_ref, o_ref):
    a1 = jnp.maximum(
        jnp.dot(r8_ref[...], w1_ref[...],
                preferred_element_type=jnp.float32) + b1_ref[...], 0.0)
    o_ref[...] = (jnp.dot(a1, w2_ref[...],
                          preferred_element_type=jnp.float32) + b2_ref[...])


def _edge_tc(rel8, w1pad, b1, w2, b2):
    return pl.pallas_call(
        _edge_body,
        grid=(_EGRID,),
        in_specs=[
            pl.BlockSpec((_EB, 8), lambda i: (i, 0)),
            pl.BlockSpec((8, _H), lambda i: (0, 0)),
            pl.BlockSpec((1, _H), lambda i: (0, 0)),
            pl.BlockSpec((_H, _H), lambda i: (0, 0)),
            pl.BlockSpec((1, _H), lambda i: (0, 0)),
        ],
        out_specs=pl.BlockSpec((_EB, _H), lambda i: (i, 0)),
        out_shape=jax.ShapeDtypeStruct((_E, _H), jnp.float32),
    )(rel8, w1pad, b1.reshape(1, _H), w2, b2.reshape(1, _H))


def _ln(t, g, b):
    mu = jnp.mean(t, axis=-1, keepdims=True)
    var = jnp.mean((t - mu) * (t - mu), axis=-1, keepdims=True)
    return (t - mu) * lax.rsqrt(var + 1e-5) * g + b


def _node_body(h_ref, a0_ref, a1_ref, w1_ref, b1_ref, w2_ref, b2_ref,
               g_ref, bb_ref, o_ref):
    z0 = a0_ref[...] + a1_ref[...] + h_ref[...]
    t = jnp.maximum(
        jnp.dot(z0, w1_ref[...], preferred_element_type=jnp.float32)
        + b1_ref[...], 0.0)
    t = (jnp.dot(t, w2_ref[...], preferred_element_type=jnp.float32)
         + b2_ref[...])
    t = _ln(t, g_ref[...], bb_ref[...])
    t = t * 0.5 * (1.0 + lax.erf(t / _SQRT2))
    o_ref[...] = t + h_ref[...]


def _node_tc(h, agg2, w1, b1, w2, b2, g, b):
    return pl.pallas_call(
        _node_body,
        grid=(_NGRID,),
        in_specs=[
            pl.BlockSpec((_NB, _H), lambda i: (i, 0)),
            pl.BlockSpec((_NB, _H), lambda i: (i, 0)),
            pl.BlockSpec((_NB, _H), lambda i: (i + _NGRID, 0)),
            pl.BlockSpec((_H, _H), lambda i: (0, 0)),
            pl.BlockSpec((1, _H), lambda i: (0, 0)),
            pl.BlockSpec((_H, _H), lambda i: (0, 0)),
            pl.BlockSpec((1, _H), lambda i: (0, 0)),
            pl.BlockSpec((1, _H), lambda i: (0, 0)),
            pl.BlockSpec((1, _H), lambda i: (0, 0)),
        ],
        out_specs=pl.BlockSpec((_NB, _H), lambda i: (i, 0)),
        out_shape=jax.ShapeDtypeStruct((_N, _H), jnp.float32),
    )(h, agg2, agg2, w1, b1.reshape(1, _H), w2, b2.reshape(1, _H),
      g.reshape(1, _H), b.reshape(1, _H))


def _final_body(h_ref, w_ref, b_ref, g_ref, bb_ref, o_ref):
    t = (jnp.dot(h_ref[...], w_ref[...], preferred_element_type=jnp.float32)
         + b_ref[...] + h_ref[...])
    o_ref[...] = _ln(t, g_ref[...], bb_ref[...])


def _final_tc(h, w, b, g, bb):
    return pl.pallas_call(
        _final_body,
        grid=(_NGRID,),
        in_specs=[
            pl.BlockSpec((_NB, _H), lambda i: (i, 0)),
            pl.BlockSpec((_H, _H), lambda i: (0, 0)),
            pl.BlockSpec((1, _H), lambda i: (0, 0)),
            pl.BlockSpec((1, _H), lambda i: (0, 0)),
            pl.BlockSpec((1, _H), lambda i: (0, 0)),
        ],
        out_specs=pl.BlockSpec((_NB, _H), lambda i: (i, 0)),
        out_shape=jax.ShapeDtypeStruct((_N, _H), jnp.float32),
    )(h, w, b.reshape(1, _H), g.reshape(1, _H), bb.reshape(1, _H))


def kernel(x, edge_index, in_w, in_b, edge_w1, edge_b1, edge_w2, edge_b2,
           conv_w1, conv_b1, conv_w2, conv_b2, norm_g, norm_b,
           final_w, final_b, out_g, out_b):
    row = edge_index[0]
    col = edge_index[1]
    posx = x[:, 0]
    posy = x[:, 1]
    zer8 = jnp.zeros((_BREL, 8), jnp.float32)
    w1pad = jnp.concatenate(
        [edge_w1, jnp.zeros((6, _H), jnp.float32)], axis=0)

    h = _h0_tc(x, in_w, in_b)
    rel8 = _rel_pos_sc(posx, posy, row, col, zer8)
    ea = _edge_tc(rel8, w1pad, edge_b1, edge_w2, edge_b2)
    eidL, rowL, colL, cnts = _bucket_sc(row, col)
    for i in range(4):
        agg4 = _msg_sc(h, ea, eidL, rowL, colL, cnts)
        agg2 = agg4[:, :, :_NPC, :].reshape(_NC * _N, _H)
        h = _node_tc(h, agg2, conv_w1[i], conv_b1[i], conv_w2[i], conv_b2[i],
                     norm_g[i], norm_b[i])
    return _final_tc(h, final_w, final_b, out_g, out_b)


# speedup vs baseline: 2.3919x; 1.0091x over previous
import math

import jax
import jax.numpy as jnp
from jax import lax
from jax.experimental import pallas as pl
from jax.experimental.pallas import tpu as pltpu
from jax.experimental.pallas import tpu_sc as plsc

_N = 10000
_E = 320000
_H = 128
_NC = 2
_NS = 16
_NPC = _N // _NS
_E2 = _E // _NC
_OCH = 2000
_NOCH = _E2 // _OCH
_FL = 2048
_BUF = 4096
_CAP = 161920
_B = 64
_NACC = 632
_BREL = 400
_EPW = _E // (_NC * _NS)
_NCHREL = _EPW // _BREL

_mesh = plsc.VectorSubcoreMesh(core_axis_name="c", subcore_axis_name="s")

_SQRT2 = math.sqrt(2.0)


def _rel_body(posx_h, posy_h, row_h, col_h, zer8_h, rel8_h,
              posx_v, posy_v, rowb, colb, rel8b, sem):
    c = lax.axis_index("c")
    s = lax.axis_index("s")
    wid = c * _NS + s
    pltpu.sync_copy(posx_h, posx_v)
    pltpu.sync_copy(posy_h, posy_v)
    pltpu.sync_copy(zer8_h, rel8b)
    base0 = wid * _EPW

    @pl.loop(0, _NCHREL)
    def _chunk(ci):
        base = pl.multiple_of(base0 + ci * _BREL, 8)
        pltpu.sync_copy(row_h.at[pl.ds(base, _BREL)], rowb)
        pltpu.sync_copy(col_h.at[pl.ds(base, _BREL)], colb)
        lane = lax.iota(jnp.int32, 16)
        zero16 = jnp.zeros((16,), jnp.int32)
        one16 = jnp.ones((16,), jnp.int32)
        for g in range(_BREL // 16):
            sl = pl.ds(g * 16, 16)
            rv = rowb[sl]
            cv = colb[sl]
            rx = plsc.load_gather(posx_v, [cv]) - plsc.load_gather(posx_v, [rv])
            ry = plsc.load_gather(posy_v, [cv]) - plsc.load_gather(posy_v, [rv])
            ridx = lane + (g * 16)
            plsc.store_scatter(rel8b, [ridx, zero16], rx)
            plsc.store_scatter(rel8b, [ridx, one16], ry)
        pltpu.sync_copy(rel8b, rel8_h.at[pl.ds(base, _BREL)])


def _rel_pos_sc(posx, posy, row, col, zer8):
    f = pl.kernel(
        _rel_body,
        out_type=jax.ShapeDtypeStruct((_E, 8), jnp.float32),
        mesh=_mesh,
        compiler_params=pltpu.CompilerParams(needs_layout_passes=False),
        scratch_types=[
            pltpu.VMEM((_N,), jnp.float32),
            pltpu.VMEM((_N,), jnp.float32),
            pltpu.VMEM((_BREL,), jnp.int32),
            pltpu.VMEM((_BREL,), jnp.int32),
            pltpu.VMEM((_BREL, 8), jnp.float32),
            pltpu.SemaphoreType.DMA,
        ],
    )
    return f(posx, posy, row, col, zer8)


def _bucket_body(row_h, col_h, eidL, rowL, colL, cnts_h,
                 crow, ccol, beid, brow, bcol, cntv, sem):
    c = lax.axis_index("c")
    s = lax.axis_index("s")
    w = c * _NS + s
    lo = s * _NPC
    base0 = c * _E2
    lane = lax.iota(jnp.int32, 16)

    def outer(i, carry):
        off, cnt = carry
        ebase = pl.multiple_of(base0 + i * _OCH, 8)
        pltpu.sync_copy(col_h.at[pl.ds(ebase, _OCH)], ccol)
        pltpu.sync_copy(row_h.at[pl.ds(ebase, _OCH)], crow)

        def inner(j, cnt):
            sl = pl.ds(j * 16, 16)
            col16 = ccol[sl]
            row16 = crow[sl]
            msk = (col16 >= lo) & (col16 < lo + _NPC)
            eid16 = jnp.full((16,), ebase, jnp.int32) + j * 16 + lane
            pos = cnt + plsc.cumsum(msk.astype(jnp.int32)) - 1
            plsc.store_scatter(beid, [pos], eid16, mask=msk)
            plsc.store_scatter(brow, [pos], row16, mask=msk)
            plsc.store_scatter(bcol, [pos], col16 - lo, mask=msk)
            return cnt + jnp.sum(msk.astype(jnp.int32))

        cnt = lax.fori_loop(0, _OCH // 16, inner, cnt)
        wb = pl.multiple_of(w * _CAP + off, 128)
        pltpu.sync_copy(beid.at[pl.ds(0, _FL)], eidL.at[pl.ds(wb, _FL)])
        pltpu.sync_copy(brow.at[pl.ds(0, _FL)], rowL.at[pl.ds(wb, _FL)])
        pltpu.sync_copy(bcol.at[pl.ds(0, _FL)], colL.at[pl.ds(wb, _FL)])
        flushed = (cnt >= _FL).astype(jnp.int32)
        fmask = jnp.full((16,), flushed, jnp.int32) > 0

        @pl.loop(0, _FL // 16)
        def _shift(j):
            d = pl.ds(j * 16, 16)
            u = pl.ds(_FL + j * 16, 16)
            beid[d] = jnp.where(fmask, beid[u], beid[d])
            brow[d] = jnp.where(fmask, brow[u], brow[d])
            bcol[d] = jnp.where(fmask, bcol[u], bcol[d])

        return (off + _FL * flushed, cnt - _FL * flushed)

    off, cnt = lax.fori_loop(0, _NOCH, outer, (0, 0))

    zero16 = jnp.zeros((16,), jnp.int32)
    spill16 = jnp.full((16,), _NPC, jnp.int32)
    for k in range(9):
        beid[pl.ds(cnt + k * 16, 16)] = zero16
        brow[pl.ds(cnt + k * 16, 16)] = zero16
        bcol[pl.ds(cnt + k * 16, 16)] = spill16
    wb = pl.multiple_of(w * _CAP + off, 128)
    pltpu.sync_copy(beid.at[pl.ds(0, _FL + 128)],
                    eidL.at[pl.ds(wb, _FL + 128)])
    pltpu.sync_copy(brow.at[pl.ds(0, _FL + 128)],
                    rowL.at[pl.ds(wb, _FL + 128)])
    pltpu.sync_copy(bcol.at[pl.ds(0, _FL + 128)],
                    colL.at[pl.ds(wb, _FL + 128)])
    nch = (off + cnt + _B - 1) // _B
    cntv[...] = jnp.full((16,), nch, jnp.int32)
    pltpu.sync_copy(cntv, cnts_h.at[pl.ds(pl.multiple_of(w * 16, 16), 16)])


def _bucket_sc(row, col):
    f = pl.kernel(
        _bucket_body,
        out_type=[
            jax.ShapeDtypeStruct((_NC * _NS * _CAP,), jnp.int32),
            jax.ShapeDtypeStruct((_NC * _NS * _CAP,), jnp.int32),
            jax.ShapeDtypeStruct((_NC * _NS * _CAP,), jnp.int32),
            jax.ShapeDtypeStruct((_NC * _NS * 16,), jnp.int32),
        ],
        mesh=_mesh,
        compiler_params=pltpu.CompilerParams(needs_layout_passes=False),
        scratch_types=[
            pltpu.VMEM((_OCH,), jnp.int32),
            pltpu.VMEM((_OCH,), jnp.int32),
            pltpu.VMEM((_BUF,), jnp.int32),
            pltpu.VMEM((_BUF,), jnp.int32),
            pltpu.VMEM((_BUF,), jnp.int32),
            pltpu.VMEM((16,), jnp.int32),
            pltpu.SemaphoreType.DMA,
        ],
    )
    return f(row, col)


_RCH = 5056


def _lane():
    return lax.iota(jnp.int32, 16)


def _off_nch(cntv_all, w):
    lane = _lane()
    c1 = plsc.load_gather(cntv_all, [lane * 16])
    c2 = plsc.load_gather(cntv_all, [lane * 16 + 256])
    o1 = jnp.sum(jnp.where(lane < jnp.minimum(w, 16), c1, 0))
    o2 = jnp.sum(jnp.where(lane + 16 < w, c2, 0))
    n1 = jnp.sum(jnp.where(lane == w, c1, 0))
    n2 = jnp.sum(jnp.where(lane + 16 == w, c2, 0))
    return o1 + o2, n1 + n2


def _perm_body(ea_h, eidL, cnts_h, eap_h,
               eidb0, eab0, eidb1, eab1, cntv_all, sem0, sem1):
    c = lax.axis_index("c")
    s = lax.axis_index("s")
    w = c * _NS + s
    slots = ((eidb0, eab0, sem0), (eidb1, eab1, sem1))
    pltpu.sync_copy(cnts_h, cntv_all)
    off, nch = _off_nch(cntv_all, w)

    def _issue(sl, ci):
        eidb, eab, sm = slots[sl]
        base = pl.multiple_of(w * _CAP, 8) + ci * _B
        pltpu.sync_copy(eidL.at[pl.ds(base, _B)], eidb)
        pltpu.async_copy(ea_h.at[eidb], eab, sm)

    def _drain_store(sl, ci):
        eidb, eab, sm = slots[sl]
        pltpu.make_async_copy(ea_h.at[eidb], eab, sm).wait()
        dst = pl.multiple_of((off + ci) * _B, 8)
        pltpu.sync_copy(eab, eap_h.at[pl.ds(dst, _B)])

    @pl.when(nch > 0)
    def _prol():
        _issue(0, 0)

    def pair(g, _):
        for b in range(2):
            ci = 2 * g + b

            @pl.when(ci + 1 < nch)
            def _i():
                _issue(1 - b, ci + 1)

            @pl.when(ci < nch)
            def _c():
                _drain_store(b, ci)

        return 0

    lax.fori_loop(0, (nch + 1) // 2, pair, 0)


def _perm_sc(ea, eidL, cnts):
    f = pl.kernel(
        _perm_body,
        out_type=jax.ShapeDtypeStruct((_RCH * _B, _H), jnp.float32),
        mesh=_mesh,
        compiler_params=pltpu.CompilerParams(needs_layout_passes=False),
        scratch_types=[
            pltpu.VMEM((_B,), jnp.int32),
            pltpu.VMEM((_B, _H), jnp.float32),
            pltpu.VMEM((_B,), jnp.int32),
            pltpu.VMEM((_B, _H), jnp.float32),
            pltpu.VMEM((_NC * _NS * 16,), jnp.int32),
            pltpu.SemaphoreType.DMA,
            pltpu.SemaphoreType.DMA,
        ],
    )
    return f(ea, eidL, cnts)


def _msg_body(h_h, eap_h, rowL, colL, cnts_h, out_h,
              rowb0, colb0, hab0, eab0,
              rowb1, colb1, hab1, eab1,
              cntv_all, acc, sem10, sem20, sem11, sem21):
    c = lax.axis_index("c")
    s = lax.axis_index("s")
    w = c * _NS + s
    zero16 = jnp.zeros((16,), jnp.float32)
    slots = ((rowb0, colb0, hab0, eab0, sem10, sem20),
             (rowb1, colb1, hab1, eab1, sem11, sem21))

    @pl.loop(0, _NACC)
    def _z(r):
        for g in range(_H // 16):
            acc[r, pl.ds(g * 16, 16)] = zero16

    pltpu.sync_copy(cnts_h, cntv_all)
    off, nch = _off_nch(cntv_all, w)

    def _issue(sl, ci):
        rowb, colb, hab, eab, s1, s2 = slots[sl]
        base = pl.multiple_of(w * _CAP, 8) + ci * _B
        pltpu.sync_copy(rowL.at[pl.ds(base, _B)], rowb)
        pltpu.sync_copy(colL.at[pl.ds(base, _B)], colb)
        pltpu.async_copy(h_h.at[rowb], hab, s1)
        src = pl.multiple_of((off + ci) * _B, 8)
        pltpu.async_copy(eap_h.at[pl.ds(src, _B)], eab, s2)

    def _drain_compute(sl, ci):
        rowb, colb, hab, eab, s1, s2 = slots[sl]
        pltpu.make_async_copy(h_h.at[rowb], hab, s1).wait()
        src = pl.multiple_of((off + ci) * _B, 8)
        pltpu.make_async_copy(eap_h.at[pl.ds(src, _B)], eab, s2).wait()

        @pl.loop(0, _B // 16)
        def _eg(j):
            col16 = colb[pl.ds(j * 16, 16)]
            for t in range(16):
                k = j * 16 + t
                cl = col16[t]
                for g in range(_H // 16):
                    sl2 = pl.ds(g * 16, 16)
                    m = jnp.maximum(hab[k, sl2] + eab[k, sl2], 0.0)
                    plsc.addupdate(acc.at[cl, sl2], m)

    @pl.when(nch > 0)
    def _prol():
        _issue(0, 0)

    def pair(g, _):
        for b in range(2):
            ci = 2 * g + b

            @pl.when(ci + 1 < nch)
            def _i():
                _issue(1 - b, ci + 1)

            @pl.when(ci < nch)
            def _c():
                _drain_compute(b, ci)

        return 0

    lax.fori_loop(0, (nch + 1) // 2, pair, 0)
    pltpu.sync_copy(acc, out_h.at[c, s])


def _msg_sc(h, eap, rowL, colL, cnts):
    f = pl.kernel(
        _msg_body,
        out_type=jax.ShapeDtypeStruct((_NC, _NS, _NACC, _H), jnp.float32),
        mesh=_mesh,
        compiler_params=pltpu.CompilerParams(needs_layout_passes=False),
        scratch_types=[
            pltpu.VMEM((_B,), jnp.int32),
            pltpu.VMEM((_B,), jnp.int32),
            pltpu.VMEM((_B, _H), jnp.float32),
            pltpu.VMEM((_B, _H), jnp.float32),
            pltpu.VMEM((_B,), jnp.int32),
            pltpu.VMEM((_B,), jnp.int32),
            pltpu.VMEM((_B, _H), jnp.float32),
            pltpu.VMEM((_B, _H), jnp.float32),
            pltpu.VMEM((_NC * _NS * 16,), jnp.int32),
            pltpu.VMEM((_NACC, _H), jnp.float32),
            pltpu.SemaphoreType.DMA,
            pltpu.SemaphoreType.DMA,
            pltpu.SemaphoreType.DMA,
            pltpu.SemaphoreType.DMA,
        ],
    )
    return f(h, eap, rowL, colL, cnts)


_NB = 400
_NGRID = _N // _NB
_EB = 640
_EGRID = _E // _EB


def _h0_body(x_ref, w_ref, b_ref, o_ref):
    o_ref[...] = (jnp.dot(x_ref[...], w_ref[...],
                          preferred_element_type=jnp.float32) + b_ref[...])


def _h0_tc(x, in_w, in_b):
    return pl.pallas_call(
        _h0_body,
        grid=(_NGRID,),
        in_specs=[
            pl.BlockSpec((_NB, _H), lambda i: (i, 0)),
            pl.BlockSpec((_H, _H), lambda i: (0, 0)),
            pl.BlockSpec((1, _H), lambda i: (0, 0)),
        ],
        out_specs=pl.BlockSpec((_NB, _H), lambda i: (i, 0)),
        out_shape=jax.ShapeDtypeStruct((_N, _H), jnp.float32),
    )(x, in_w, in_b.reshape(1, _H))


def _edge_body(r8_ref, w1_ref, b1_ref, w2_ref, b2_ref, o_ref):
    a1 = jnp.maximum(
        jnp.dot(r8_ref[...], w1_ref[...],
                preferred_element_type=jnp.float32) + b1_ref[...], 0.0)
    o_ref[...] = (jnp.dot(a1, w2_ref[...],
                          preferred_element_type=jnp.float32) + b2_ref[...])


def _edge_tc(rel8, w1pad, b1, w2, b2):
    return pl.pallas_call(
        _edge_body,
        grid=(_EGRID,),
        in_specs=[
            pl.BlockSpec((_EB, 8), lambda i: (i, 0)),
            pl.BlockSpec((8, _H), lambda i: (0, 0)),
            pl.BlockSpec((1, _H), lambda i: (0, 0)),
            pl.BlockSpec((_H, _H), lambda i: (0, 0)),
            pl.BlockSpec((1, _H), lambda i: (0, 0)),
        ],
        out_specs=pl.BlockSpec((_EB, _H), lambda i: (i, 0)),
        out_shape=jax.ShapeDtypeStruct((_E, _H), jnp.float32),
    )(rel8, w1pad, b1.reshape(1, _H), w2, b2.reshape(1, _H))


def _ln(t, g, b):
    mu = jnp.mean(t, axis=-1, keepdims=True)
    var = jnp.mean((t - mu) * (t - mu), axis=-1, keepdims=True)
    return (t - mu) * lax.rsqrt(var + 1e-5) * g + b


def _node_body(h_ref, a0_ref, a1_ref, w1_ref, b1_ref, w2_ref, b2_ref,
               g_ref, bb_ref, o_ref):
    z0 = a0_ref[...] + a1_ref[...] + h_ref[...]
    t = jnp.maximum(
        jnp.dot(z0, w1_ref[...], preferred_element_type=jnp.float32)
        + b1_ref[...], 0.0)
    t = (jnp.dot(t, w2_ref[...], preferred_element_type=jnp.float32)
         + b2_ref[...])
    t = _ln(t, g_ref[...], bb_ref[...])
    t = t * 0.5 * (1.0 + lax.erf(t / _SQRT2))
    o_ref[...] = t + h_ref[...]


def _node_tc(h, agg2, w1, b1, w2, b2, g, b):
    return pl.pallas_call(
        _node_body,
        grid=(_NGRID,),
        in_specs=[
            pl.BlockSpec((_NB, _H), lambda i: (i, 0)),
            pl.BlockSpec((_NB, _H), lambda i: (i, 0)),
            pl.BlockSpec((_NB, _H), lambda i: (i + _NGRID, 0)),
            pl.BlockSpec((_H, _H), lambda i: (0, 0)),
            pl.BlockSpec((1, _H), lambda i: (0, 0)),
            pl.BlockSpec((_H, _H), lambda i: (0, 0)),
            pl.BlockSpec((1, _H), lambda i: (0, 0)),
            pl.BlockSpec((1, _H), lambda i: (0, 0)),
            pl.BlockSpec((1, _H), lambda i: (0, 0)),
        ],
        out_specs=pl.BlockSpec((_NB, _H), lambda i: (i, 0)),
        out_shape=jax.ShapeDtypeStruct((_N, _H), jnp.float32),
    )(h, agg2, agg2, w1, b1.reshape(1, _H), w2, b2.reshape(1, _H),
      g.reshape(1, _H), b.reshape(1, _H))


def _final_body(h_ref, w_ref, b_ref, g_ref, bb_ref, o_ref):
    t = (jnp.dot(h_ref[...], w_ref[...], preferred_element_type=jnp.float32)
         + b_ref[...] + h_ref[...])
    o_ref[...] = _ln(t, g_ref[...], bb_ref[...])


def _final_tc(h, w, b, g, bb):
    return pl.pallas_call(
        _final_body,
        grid=(_NGRID,),
        in_specs=[
            pl.BlockSpec((_NB, _H), lambda i: (i, 0)),
            pl.BlockSpec((_H, _H), lambda i: (0, 0)),
            pl.BlockSpec((1, _H), lambda i: (0, 0)),
            pl.BlockSpec((1, _H), lambda i: (0, 0)),
            pl.BlockSpec((1, _H), lambda i: (0, 0)),
        ],
        out_specs=pl.BlockSpec((_NB, _H), lambda i: (i, 0)),
        out_shape=jax.ShapeDtypeStruct((_N, _H), jnp.float32),
    )(h, w, b.reshape(1, _H), g.reshape(1, _H), bb.reshape(1, _H))


def kernel(x, edge_index, in_w, in_b, edge_w1, edge_b1, edge_w2, edge_b2,
           conv_w1, conv_b1, conv_w2, conv_b2, norm_g, norm_b,
           final_w, final_b, out_g, out_b):
    row = edge_index[0]
    col = edge_index[1]
    posx = x[:, 0]
    posy = x[:, 1]
    zer8 = jnp.zeros((_BREL, 8), jnp.float32)
    w1pad = jnp.concatenate(
        [edge_w1, jnp.zeros((6, _H), jnp.float32)], axis=0)

    h = _h0_tc(x, in_w, in_b)
    rel8 = _rel_pos_sc(posx, posy, row, col, zer8)
    ea = _edge_tc(rel8, w1pad, edge_b1, edge_w2, edge_b2)
    eidL, rowL, colL, cnts = _bucket_sc(row, col)
    eap = _perm_sc(ea, eidL, cnts)
    for i in range(4):
        agg4 = _msg_sc(h, eap, rowL, colL, cnts)
        agg2 = agg4[:, :, :_NPC, :].reshape(_NC * _N, _H)
        h = _node_tc(h, agg2, conv_w1[i], conv_b1[i], conv_w2[i], conv_b2[i],
                     norm_g[i], norm_b[i])
    return _final_tc(h, final_w, final_b, out_g, out_b)


# speedup vs baseline: 3.7412x; 1.5641x over previous
import math

import jax
import jax.numpy as jnp
from jax import lax
from jax.experimental import pallas as pl
from jax.experimental.pallas import tpu as pltpu
from jax.experimental.pallas import tpu_sc as plsc

_N = 10000
_E = 320000
_H = 128
_NC = 2
_NS = 16
_NPC = _N // _NS
_E2 = _E // _NC
_OCH = 2000
_NOCH = _E2 // _OCH
_FL = 2048
_BUF = 4096
_CAP = 161920
_B = 64
_NACC = 632
_BREL = 400
_EPW = _E // (_NC * _NS)
_NCHREL = _EPW // _BREL

_mesh = plsc.VectorSubcoreMesh(core_axis_name="c", subcore_axis_name="s")

_SQRT2 = math.sqrt(2.0)


def _rel_body(posx_h, posy_h, row_h, col_h, zer8_h, rel8_h,
              posx_v, posy_v, rowb, colb, rel8b, sem):
    c = lax.axis_index("c")
    s = lax.axis_index("s")
    wid = c * _NS + s
    pltpu.sync_copy(posx_h, posx_v)
    pltpu.sync_copy(posy_h, posy_v)
    pltpu.sync_copy(zer8_h, rel8b)
    base0 = wid * _EPW

    @pl.loop(0, _NCHREL)
    def _chunk(ci):
        base = pl.multiple_of(base0 + ci * _BREL, 8)
        pltpu.sync_copy(row_h.at[pl.ds(base, _BREL)], rowb)
        pltpu.sync_copy(col_h.at[pl.ds(base, _BREL)], colb)
        lane = lax.iota(jnp.int32, 16)
        zero16 = jnp.zeros((16,), jnp.int32)
        one16 = jnp.ones((16,), jnp.int32)
        for g in range(_BREL // 16):
            sl = pl.ds(g * 16, 16)
            rv = rowb[sl]
            cv = colb[sl]
            rx = plsc.load_gather(posx_v, [cv]) - plsc.load_gather(posx_v, [rv])
            ry = plsc.load_gather(posy_v, [cv]) - plsc.load_gather(posy_v, [rv])
            ridx = lane + (g * 16)
            plsc.store_scatter(rel8b, [ridx, zero16], rx)
            plsc.store_scatter(rel8b, [ridx, one16], ry)
        pltpu.sync_copy(rel8b, rel8_h.at[pl.ds(base, _BREL)])


def _rel_pos_sc(posx, posy, row, col, zer8):
    f = pl.kernel(
        _rel_body,
        out_type=jax.ShapeDtypeStruct((_E, 8), jnp.float32),
        mesh=_mesh,
        compiler_params=pltpu.CompilerParams(needs_layout_passes=False),
        scratch_types=[
            pltpu.VMEM((_N,), jnp.float32),
            pltpu.VMEM((_N,), jnp.float32),
            pltpu.VMEM((_BREL,), jnp.int32),
            pltpu.VMEM((_BREL,), jnp.int32),
            pltpu.VMEM((_BREL, 8), jnp.float32),
            pltpu.SemaphoreType.DMA,
        ],
    )
    return f(posx, posy, row, col, zer8)


def _bucket_body(row_h, col_h, eidL, rowL, colL, cnts_h,
                 crow, ccol, beid, brow, bcol, cntv, sem):
    c = lax.axis_index("c")
    s = lax.axis_index("s")
    w = c * _NS + s
    lo = s * _NPC
    base0 = c * _E2
    lane = lax.iota(jnp.int32, 16)

    def outer(i, carry):
        off, cnt = carry
        ebase = pl.multiple_of(base0 + i * _OCH, 8)
        pltpu.sync_copy(col_h.at[pl.ds(ebase, _OCH)], ccol)
        pltpu.sync_copy(row_h.at[pl.ds(ebase, _OCH)], crow)

        def inner(j, cnt):
            sl = pl.ds(j * 16, 16)
            col16 = ccol[sl]
            row16 = crow[sl]
            msk = (col16 >= lo) & (col16 < lo + _NPC)
            eid16 = jnp.full((16,), ebase, jnp.int32) + j * 16 + lane
            pos = cnt + plsc.cumsum(msk.astype(jnp.int32)) - 1
            plsc.store_scatter(beid, [pos], eid16, mask=msk)
            plsc.store_scatter(brow, [pos], row16, mask=msk)
            plsc.store_scatter(bcol, [pos], col16 - lo, mask=msk)
            return cnt + jnp.sum(msk.astype(jnp.int32))

        cnt = lax.fori_loop(0, _OCH // 16, inner, cnt)
        wb = pl.multiple_of(w * _CAP + off, 128)
        pltpu.sync_copy(beid.at[pl.ds(0, _FL)], eidL.at[pl.ds(wb, _FL)])
        pltpu.sync_copy(brow.at[pl.ds(0, _FL)], rowL.at[pl.ds(wb, _FL)])
        pltpu.sync_copy(bcol.at[pl.ds(0, _FL)], colL.at[pl.ds(wb, _FL)])
        flushed = (cnt >= _FL).astype(jnp.int32)
        fmask = jnp.full((16,), flushed, jnp.int32) > 0

        @pl.loop(0, _FL // 16)
        def _shift(j):
            d = pl.ds(j * 16, 16)
            u = pl.ds(_FL + j * 16, 16)
            beid[d] = jnp.where(fmask, beid[u], beid[d])
            brow[d] = jnp.where(fmask, brow[u], brow[d])
            bcol[d] = jnp.where(fmask, bcol[u], bcol[d])

        return (off + _FL * flushed, cnt - _FL * flushed)

    off, cnt = lax.fori_loop(0, _NOCH, outer, (0, 0))

    zero16 = jnp.zeros((16,), jnp.int32)
    spill16 = jnp.full((16,), _NPC, jnp.int32)
    for k in range(9):
        beid[pl.ds(cnt + k * 16, 16)] = zero16
        brow[pl.ds(cnt + k * 16, 16)] = zero16
        bcol[pl.ds(cnt + k * 16, 16)] = spill16
    wb = pl.multiple_of(w * _CAP + off, 128)
    pltpu.sync_copy(beid.at[pl.ds(0, _FL + 128)],
                    eidL.at[pl.ds(wb, _FL + 128)])
    pltpu.sync_copy(brow.at[pl.ds(0, _FL + 128)],
                    rowL.at[pl.ds(wb, _FL + 128)])
    pltpu.sync_copy(bcol.at[pl.ds(0, _FL + 128)],
                    colL.at[pl.ds(wb, _FL + 128)])
    nch = (off + cnt + _B - 1) // _B
    cntv[...] = jnp.full((16,), nch, jnp.int32)
    pltpu.sync_copy(cntv, cnts_h.at[pl.ds(pl.multiple_of(w * 16, 16), 16)])


def _bucket_sc(row, col):
    f = pl.kernel(
        _bucket_body,
        out_type=[
            jax.ShapeDtypeStruct((_NC * _NS * _CAP,), jnp.int32),
            jax.ShapeDtypeStruct((_NC * _NS * _CAP,), jnp.int32),
            jax.ShapeDtypeStruct((_NC * _NS * _CAP,), jnp.int32),
            jax.ShapeDtypeStruct((_NC * _NS * 16,), jnp.int32),
        ],
        mesh=_mesh,
        compiler_params=pltpu.CompilerParams(needs_layout_passes=False),
        scratch_types=[
            pltpu.VMEM((_OCH,), jnp.int32),
            pltpu.VMEM((_OCH,), jnp.int32),
            pltpu.VMEM((_BUF,), jnp.int32),
            pltpu.VMEM((_BUF,), jnp.int32),
            pltpu.VMEM((_BUF,), jnp.int32),
            pltpu.VMEM((16,), jnp.int32),
            pltpu.SemaphoreType.DMA,
        ],
    )
    return f(row, col)


_RCH = 5056


def _lane():
    return lax.iota(jnp.int32, 16)


def _off_nch(cntv_all, w):
    lane = _lane()
    c1 = plsc.load_gather(cntv_all, [lane * 16])
    c2 = plsc.load_gather(cntv_all, [lane * 16 + 256])
    o1 = jnp.sum(jnp.where(lane < jnp.minimum(w, 16), c1, 0))
    o2 = jnp.sum(jnp.where(lane + 16 < w, c2, 0))
    n1 = jnp.sum(jnp.where(lane == w, c1, 0))
    n2 = jnp.sum(jnp.where(lane + 16 == w, c2, 0))
    return o1 + o2, n1 + n2


def _perm_body(ea_h, eidL, cnts_h, eap_h,
               eidb0, eab0, eidb1, eab1, cntv_all, sem0, sem1):
    c = lax.axis_index("c")
    s = lax.axis_index("s")
    w = c * _NS + s
    slots = ((eidb0, eab0, sem0), (eidb1, eab1, sem1))
    pltpu.sync_copy(cnts_h, cntv_all)
    off, nch = _off_nch(cntv_all, w)

    def _issue(sl, ci):
        eidb, eab, sm = slots[sl]
        base = pl.multiple_of(w * _CAP, 8) + ci * _B
        pltpu.sync_copy(eidL.at[pl.ds(base, _B)], eidb)
        pltpu.async_copy(ea_h.at[eidb], eab, sm)

    def _drain_store(sl, ci):
        eidb, eab, sm = slots[sl]
        pltpu.make_async_copy(ea_h.at[eidb], eab, sm).wait()
        dst = pl.multiple_of((off + ci) * _B, 8)
        pltpu.sync_copy(eab, eap_h.at[pl.ds(dst, _B)])

    @pl.when(nch > 0)
    def _prol():
        _issue(0, 0)

    def pair(g, _):
        for b in range(2):
            ci = 2 * g + b

            @pl.when(ci + 1 < nch)
            def _i():
                _issue(1 - b, ci + 1)

            @pl.when(ci < nch)
            def _c():
                _drain_store(b, ci)

        return 0

    lax.fori_loop(0, (nch + 1) // 2, pair, 0)


def _perm_sc(ea, eidL, cnts):
    f = pl.kernel(
        _perm_body,
        out_type=jax.ShapeDtypeStruct((_RCH * _B, _H), jnp.float32),
        mesh=_mesh,
        compiler_params=pltpu.CompilerParams(needs_layout_passes=False),
        scratch_types=[
            pltpu.VMEM((_B,), jnp.int32),
            pltpu.VMEM((_B, _H), jnp.float32),
            pltpu.VMEM((_B,), jnp.int32),
            pltpu.VMEM((_B, _H), jnp.float32),
            pltpu.VMEM((_NC * _NS * 16,), jnp.int32),
            pltpu.SemaphoreType.DMA,
            pltpu.SemaphoreType.DMA,
        ],
    )
    return f(ea, eidL, cnts)


def _msg_body(h_h, eap_h, rowL, colL, cnts_h, out_h,
              rowb0, colb0, hab0, eab0,
              rowb1, colb1, hab1, eab1,
              cntv_all, acc, sem10, sem20, sem30, sem11, sem21, sem31):
    c = lax.axis_index("c")
    s = lax.axis_index("s")
    w = c * _NS + s
    zero16 = jnp.zeros((16,), jnp.float32)
    slots = ((rowb0, colb0, hab0, eab0, sem10, sem20, sem30),
             (rowb1, colb1, hab1, eab1, sem11, sem21, sem31))

    @pl.loop(0, _B)
    def _zb(r):
        for g in range(_H // 16):
            hab0[r, pl.ds(g * 16, 16)] = zero16

    abase = s * _NACC
    for r in range(9):
        pltpu.sync_copy(hab0, acc.at[pl.ds(abase + r * _B, _B)])
    pltpu.sync_copy(hab0.at[pl.ds(0, 56)],
                    acc.at[pl.ds(abase + 9 * _B, 56)])

    pltpu.sync_copy(cnts_h, cntv_all)
    off, nch = _off_nch(cntv_all, w)

    def _issue(sl, ci):
        rowb, colb, hab, eab, s1, s2, s3 = slots[sl]

        @pl.when(ci >= 2)
        def _w():
            pltpu.make_async_copy(hab, acc.at[colb], s3).wait()

        base = pl.multiple_of(w * _CAP, 8) + ci * _B
        pltpu.sync_copy(rowL.at[pl.ds(base, _B)], rowb)
        pltpu.sync_copy(colL.at[pl.ds(base, _B)], colb)
        for q in range(_B // 16):
            colb[pl.ds(q * 16, 16)] = colb[pl.ds(q * 16, 16)] + abase
        pltpu.async_copy(h_h.at[rowb], hab, s1)
        src = pl.multiple_of((off + ci) * _B, 8)
        pltpu.async_copy(eap_h.at[pl.ds(src, _B)], eab, s2)

    def _drain_compute(sl, ci):
        rowb, colb, hab, eab, s1, s2, s3 = slots[sl]
        pltpu.make_async_copy(h_h.at[rowb], hab, s1).wait()
        src = pl.multiple_of((off + ci) * _B, 8)
        pltpu.make_async_copy(eap_h.at[pl.ds(src, _B)], eab, s2).wait()

        @pl.loop(0, _B)
        def _eg(k):
            for g in range(_H // 16):
                sl2 = pl.ds(g * 16, 16)
                hab[k, sl2] = jnp.maximum(hab[k, sl2] + eab[k, sl2], 0.0)

        pltpu.async_copy(hab, acc.at[colb], s3, add=True)

    @pl.when(nch > 0)
    def _prol():
        _issue(0, 0)

    def pair(g, _):
        for b in range(2):
            ci = 2 * g + b

            @pl.when(ci + 1 < nch)
            def _i():
                _issue(1 - b, ci + 1)

            @pl.when(ci < nch)
            def _c():
                _drain_compute(b, ci)

        return 0

    lax.fori_loop(0, (nch + 1) // 2, pair, 0)

    @pl.when(nch >= 1)
    def _d0():
        pltpu.make_async_copy(hab0, acc.at[colb0], sem30).wait()

    @pl.when(nch >= 2)
    def _d1():
        pltpu.make_async_copy(hab1, acc.at[colb1], sem31).wait()

    pltpu.sync_copy(acc.at[pl.ds(abase, _NACC)], out_h.at[c, s])


def _msg_sc(h, eap, rowL, colL, cnts):
    f = pl.kernel(
        _msg_body,
        out_type=jax.ShapeDtypeStruct((_NC, _NS, _NACC, _H), jnp.float32),
        mesh=_mesh,
        compiler_params=pltpu.CompilerParams(needs_layout_passes=False),
        scratch_types=[
            pltpu.VMEM((_B,), jnp.int32),
            pltpu.VMEM((_B,), jnp.int32),
            pltpu.VMEM((_B, _H), jnp.float32),
            pltpu.VMEM((_B, _H), jnp.float32),
            pltpu.VMEM((_B,), jnp.int32),
            pltpu.VMEM((_B,), jnp.int32),
            pltpu.VMEM((_B, _H), jnp.float32),
            pltpu.VMEM((_B, _H), jnp.float32),
            pltpu.VMEM((_NC * _NS * 16,), jnp.int32),
            pltpu.VMEM_SHARED((_NS * _NACC, _H), jnp.float32),
            pltpu.SemaphoreType.DMA,
            pltpu.SemaphoreType.DMA,
            pltpu.SemaphoreType.DMA,
            pltpu.SemaphoreType.DMA,
            pltpu.SemaphoreType.DMA,
            pltpu.SemaphoreType.DMA,
        ],
    )
    return f(h, eap, rowL, colL, cnts)


_NB = 400
_NGRID = _N // _NB
_EB = 640
_EGRID = _E // _EB


def _h0_body(x_ref, w_ref, b_ref, o_ref):
    o_ref[...] = (jnp.dot(x_ref[...], w_ref[...],
                          preferred_element_type=jnp.float32) + b_ref[...])


def _h0_tc(x, in_w, in_b):
    return pl.pallas_call(
        _h0_body,
        grid=(_NGRID,),
        in_specs=[
            pl.BlockSpec((_NB, _H), lambda i: (i, 0)),
            pl.BlockSpec((_H, _H), lambda i: (0, 0)),
            pl.BlockSpec((1, _H), lambda i: (0, 0)),
        ],
        out_specs=pl.BlockSpec((_NB, _H), lambda i: (i, 0)),
        out_shape=jax.ShapeDtypeStruct((_N, _H), jnp.float32),
    )(x, in_w, in_b.reshape(1, _H))


def _edge_body(r8_ref, w1_ref, b1_ref, w2_ref, b2_ref, o_ref):
    a1 = jnp.maximum(
        jnp.dot(r8_ref[...], w1_ref[...],
                preferred_element_type=jnp.float32) + b1_ref[...], 0.0)
    o_ref[...] = (jnp.dot(a1, w2_ref[...],
                          preferred_element_type=jnp.float32) + b2_ref[...])


def _edge_tc(rel8, w1pad, b1, w2, b2):
    return pl.pallas_call(
        _edge_body,
        grid=(_EGRID,),
        in_specs=[
            pl.BlockSpec((_EB, 8), lambda i: (i, 0)),
            pl.BlockSpec((8, _H), lambda i: (0, 0)),
            pl.BlockSpec((1, _H), lambda i: (0, 0)),
            pl.BlockSpec((_H, _H), lambda i: (0, 0)),
            pl.BlockSpec((1, _H), lambda i: (0, 0)),
        ],
        out_specs=pl.BlockSpec((_EB, _H), lambda i: (i, 0)),
        out_shape=jax.ShapeDtypeStruct((_E, _H), jnp.float32),
    )(rel8, w1pad, b1.reshape(1, _H), w2, b2.reshape(1, _H))


def _ln(t, g, b):
    mu = jnp.mean(t, axis=-1, keepdims=True)
    var = jnp.mean((t - mu) * (t - mu), axis=-1, keepdims=True)
    return (t - mu) * lax.rsqrt(var + 1e-5) * g + b


def _node_body(h_ref, a0_ref, a1_ref, w1_ref, b1_ref, w2_ref, b2_ref,
               g_ref, bb_ref, o_ref):
    z0 = a0_ref[...] + a1_ref[...] + h_ref[...]
    t = jnp.maximum(
        jnp.dot(z0, w1_ref[...], preferred_element_type=jnp.float32)
        + b1_ref[...], 0.0)
    t = (jnp.dot(t, w2_ref[...], preferred_element_type=jnp.float32)
         + b2_ref[...])
    t = _ln(t, g_ref[...], bb_ref[...])
    t = t * 0.5 * (1.0 + lax.erf(t / _SQRT2))
    o_ref[...] = t + h_ref[...]


def _node_tc(h, agg2, w1, b1, w2, b2, g, b):
    return pl.pallas_call(
        _node_body,
        grid=(_NGRID,),
        in_specs=[
            pl.BlockSpec((_NB, _H), lambda i: (i, 0)),
            pl.BlockSpec((_NB, _H), lambda i: (i, 0)),
            pl.BlockSpec((_NB, _H), lambda i: (i + _NGRID, 0)),
            pl.BlockSpec((_H, _H), lambda i: (0, 0)),
            pl.BlockSpec((1, _H), lambda i: (0, 0)),
            pl.BlockSpec((_H, _H), lambda i: (0, 0)),
            pl.BlockSpec((1, _H), lambda i: (0, 0)),
            pl.BlockSpec((1, _H), lambda i: (0, 0)),
            pl.BlockSpec((1, _H), lambda i: (0, 0)),
        ],
        out_specs=pl.BlockSpec((_NB, _H), lambda i: (i, 0)),
        out_shape=jax.ShapeDtypeStruct((_N, _H), jnp.float32),
    )(h, agg2, agg2, w1, b1.reshape(1, _H), w2, b2.reshape(1, _H),
      g.reshape(1, _H), b.reshape(1, _H))


def _final_body(h_ref, w_ref, b_ref, g_ref, bb_ref, o_ref):
    t = (jnp.dot(h_ref[...], w_ref[...], preferred_element_type=jnp.float32)
         + b_ref[...] + h_ref[...])
    o_ref[...] = _ln(t, g_ref[...], bb_ref[...])


def _final_tc(h, w, b, g, bb):
    return pl.pallas_call(
        _final_body,
        grid=(_NGRID,),
        in_specs=[
            pl.BlockSpec((_NB, _H), lambda i: (i, 0)),
            pl.BlockSpec((_H, _H), lambda i: (0, 0)),
            pl.BlockSpec((1, _H), lambda i: (0, 0)),
            pl.BlockSpec((1, _H), lambda i: (0, 0)),
            pl.BlockSpec((1, _H), lambda i: (0, 0)),
        ],
        out_specs=pl.BlockSpec((_NB, _H), lambda i: (i, 0)),
        out_shape=jax.ShapeDtypeStruct((_N, _H), jnp.float32),
    )(h, w, b.reshape(1, _H), g.reshape(1, _H), bb.reshape(1, _H))


def kernel(x, edge_index, in_w, in_b, edge_w1, edge_b1, edge_w2, edge_b2,
           conv_w1, conv_b1, conv_w2, conv_b2, norm_g, norm_b,
           final_w, final_b, out_g, out_b):
    row = edge_index[0]
    col = edge_index[1]
    posx = x[:, 0]
    posy = x[:, 1]
    zer8 = jnp.zeros((_BREL, 8), jnp.float32)
    w1pad = jnp.concatenate(
        [edge_w1, jnp.zeros((6, _H), jnp.float32)], axis=0)

    h = _h0_tc(x, in_w, in_b)
    rel8 = _rel_pos_sc(posx, posy, row, col, zer8)
    ea = _edge_tc(rel8, w1pad, edge_b1, edge_w2, edge_b2)
    eidL, rowL, colL, cnts = _bucket_sc(row, col)
    eap = _perm_sc(ea, eidL, cnts)
    for i in range(4):
        agg4 = _msg_sc(h, eap, rowL, colL, cnts)
        agg2 = agg4[:, :, :_NPC, :].reshape(_NC * _N, _H)
        h = _node_tc(h, agg2, conv_w1[i], conv_b1[i], conv_w2[i], conv_b2[i],
                     norm_g[i], norm_b[i])
    return _final_tc(h, final_w, final_b, out_g, out_b)
